# baseline (speedup 1.0000x reference)
import functools

import jax
import jax.numpy as jnp
from jax import lax
from jax.experimental import pallas as pl
from jax.experimental.pallas import tpu as pltpu
from jax.experimental.pallas import tpu_sc as plsc

SEQ_LEN = 8192
EMBED_DIM = 1024

_info = plsc.get_sparse_core_info()
_NC, _NS = _info.num_cores, _info.num_subcores
_NW = _NC * _NS
_ROWS_PER_C = SEQ_LEN // _NC
_CH = 256
_NCHUNK = _ROWS_PER_C // _CH
_NBUF = 3

_mesh = plsc.VectorSubcoreMesh(core_axis_name="c", subcore_axis_name="s")


@functools.partial(
    pl.kernel,
    mesh=_mesh,
    out_type=jax.ShapeDtypeStruct((SEQ_LEN, EMBED_DIM), jnp.float32),
    scratch_types=[
        pltpu.VMEM_SHARED((_NBUF, _CH, EMBED_DIM), jnp.float32),
        pltpu.SemaphoreType.DMA((_NBUF,)),
        pltpu.SemaphoreType.DMA((_NBUF,)),
    ],
)
def _copy_rows(emb_hbm, out_hbm, stage, in_sems, out_sems):
    c = lax.axis_index("c")
    s = lax.axis_index("s")
    base = c * _ROWS_PER_C

    def in_copy(i):
        return pltpu.make_async_copy(
            emb_hbm.at[pl.ds(base + i * _CH, _CH)],
            stage.at[i % _NBUF],
            in_sems.at[i % _NBUF],
        )

    def out_copy(i):
        return pltpu.make_async_copy(
            stage.at[i % _NBUF],
            out_hbm.at[pl.ds(base + i * _CH, _CH)],
            out_sems.at[i % _NBUF],
        )

    @pl.when(s == 0)
    def _():
        for i in range(_NBUF):
            in_copy(i).start()
        out_waited = set()
        for i in range(_NCHUNK):
            in_copy(i).wait()
            out_copy(i).start()
            if i >= 1 and i + _NBUF - 1 < _NCHUNK:
                out_copy(i - 1).wait()
                out_waited.add(i - 1)
                in_copy(i + _NBUF - 1).start()
        for i in range(_NCHUNK):
            if i not in out_waited:
                out_copy(i).wait()


def kernel(x, embedding):
    del x
    return _copy_rows(embedding)


# speedup vs baseline: 1.1423x; 1.1423x over previous
import functools

import jax
import jax.numpy as jnp
from jax import lax
from jax.experimental import pallas as pl
from jax.experimental.pallas import tpu as pltpu
from jax.experimental.pallas import tpu_sc as plsc

SEQ_LEN = 8192
EMBED_DIM = 1024

_info = plsc.get_sparse_core_info()
_NC, _NS = _info.num_cores, _info.num_subcores
_NW = _NC * _NS
_ROWS_PER_W = SEQ_LEN // _NW
_CH = 32
_NCHUNK = _ROWS_PER_W // _CH

_mesh = plsc.VectorSubcoreMesh(core_axis_name="c", subcore_axis_name="s")


def _pipeline_ops(n, nbuf=2):
    ops = [("si", i) for i in range(nbuf)]
    ops += [("wi", 0), ("so", 0)]
    for i in range(1, n):
        ops += [("wi", i), ("so", i), ("wo", i - 1)]
        if i + 1 < n:
            ops.append(("si", i + 1))
    ops.append(("wo", n - 1))
    return ops


@functools.partial(
    pl.kernel,
    mesh=_mesh,
    out_type=jax.ShapeDtypeStruct((SEQ_LEN, EMBED_DIM), jnp.float32),
    scratch_types=[
        pltpu.VMEM((2, _CH, EMBED_DIM), jnp.float32),
        pltpu.VMEM_SHARED((_NS, 2, _CH, EMBED_DIM), jnp.float32),
        pltpu.SemaphoreType.DMA((2,)),
        pltpu.SemaphoreType.DMA((2,)),
        pltpu.SemaphoreType.DMA((2,)),
        pltpu.SemaphoreType.DMA((2,)),
    ],
)
def _copy_rows(emb_hbm, out_hbm, stage_t, stage_s, in_t, out_t, in_s, out_s):
    c = lax.axis_index("c")
    s = lax.axis_index("s")
    base = (s * _NC + c) * _ROWS_PER_W

    def mk(stage, in_sems, out_sems, off):
        def in_copy(i):
            return pltpu.make_async_copy(
                emb_hbm.at[pl.ds(base + (off + i) * _CH, _CH)],
                stage.at[i % 2],
                in_sems.at[i % 2],
            )

        def out_copy(i):
            return pltpu.make_async_copy(
                stage.at[i % 2],
                out_hbm.at[pl.ds(base + (off + i) * _CH, _CH)],
                out_sems.at[i % 2],
            )

        return in_copy, out_copy

    half = _NCHUNK // 2
    a_in, a_out = mk(stage_t, in_t, out_t, 0)
    b_in, b_out = mk(stage_s.at[s], in_s, out_s, half)
    ops_a = _pipeline_ops(half)
    ops_b = _pipeline_ops(half)
    run = {
        "si": lambda f, i: f[0](i).start(),
        "so": lambda f, i: f[1](i).start(),
        "wi": lambda f, i: f[0](i).wait(),
        "wo": lambda f, i: f[1](i).wait(),
    }
    for j in range(max(len(ops_a), len(ops_b))):
        if j < len(ops_a):
            k, i = ops_a[j]
            run[k]((a_in, a_out), i)
        if j < len(ops_b):
            k, i = ops_b[j]
            run[k]((b_in, b_out), i)


def kernel(x, embedding):
    del x
    return _copy_rows(embedding)


# speedup vs baseline: 1.9712x; 1.7256x over previous
import jax
import jax.numpy as jnp
from jax.experimental import pallas as pl

SEQ_LEN = 8192
EMBED_DIM = 1024
_BR = 512


def _body(in_ref, o_ref):
    o_ref[...] = in_ref[...]


def kernel(x, embedding):
    del x
    return pl.pallas_call(
        _body,
        grid=(SEQ_LEN // _BR,),
        in_specs=[pl.BlockSpec((_BR, EMBED_DIM), lambda i: (i, 0))],
        out_specs=pl.BlockSpec((_BR, EMBED_DIM), lambda i: (i, 0)),
        out_shape=jax.ShapeDtypeStruct((SEQ_LEN, EMBED_DIM), jnp.float32),
    )(embedding)
